# Initial kernel scaffold; baseline (speedup 1.0000x reference)
import jax
import jax.numpy as jnp
from jax.experimental import pallas as pl


def kernel(input, level_weight, channels_weight):
    raise NotImplementedError("write your pallas kernel here")


# speedup vs baseline: 3.6404x; 3.6404x over previous
import functools

import jax
import jax.numpy as jnp
from jax import lax
from jax.experimental import pallas as pl
from jax.experimental.pallas import tpu as pltpu
from jax.experimental.pallas import tpu_sc as plsc

NUM_LEVELS = 201
LOW = -100.0
HIGH = 100.0
B, T, C, D = 1024, 20, 26, 64
N = 4
NWIN = T - (N - 1)

NC, NS = 2, 16
NW = NC * NS
BPW = B // NW
RPB = T * C
NSTREAM, SLEN = 5, 104


def _prologue(level_weight, channels_weight):
    def body(lw_ref, cw_ref, tab_ref):
        prod = lw_ref[...][None, :, :] * cw_ref[...][:, None, :]
        tab_ref[...] = jnp.concatenate([prod, jnp.zeros_like(prod)], axis=-1)

    return pl.pallas_call(
        body,
        out_shape=jax.ShapeDtypeStruct((C, NUM_LEVELS, 2 * D), jnp.float32),
    )(level_weight, channels_weight)


def _sc_encode(idx, tab):
    mesh = plsc.VectorSubcoreMesh(core_axis_name="c", subcore_axis_name="s")

    @functools.partial(
        pl.kernel,
        mesh=mesh,
        out_type=jax.ShapeDtypeStruct((B, D), jnp.float32),
        scratch_types=[
            pltpu.VMEM((BPW * NSTREAM, SLEN), jnp.int32),
            pltpu.VMEM((RPB, 2 * D), jnp.float32),
            pltpu.VMEM((T, 2 * D), jnp.float32),
            pltpu.VMEM((BPW, D), jnp.float32),
            pltpu.SemaphoreType.DMA,
        ],
    )
    def k(idx_hbm, tab_hbm, out_hbm, idx_v, rows_v, sdup_v, outb_v, sem):
        wid = lax.axis_index("s") * NC + lax.axis_index("c")
        pltpu.sync_copy(idx_hbm.at[wid], idx_v)

        zeros = jnp.zeros((16,), jnp.float32)

        def b_body(b, _):
            handles = []
            for s in range(NSTREAM):
                handles.append(
                    pltpu.async_copy(
                        tab_hbm.at[idx_v.at[b * NSTREAM + s]],
                        rows_v.at[pl.ds(s * SLEN, SLEN)],
                        sem,
                    )
                )
            for h in handles:
                h.wait()

            def t_body(t, _):
                def c_body(c, acc):
                    r = t * C + c
                    return tuple(
                        acc[w] + rows_v[r, pl.ds(16 * w, 16)] for w in range(4)
                    )

                acc = lax.fori_loop(0, C, c_body, (zeros,) * 4)
                for w in range(4):
                    sdup_v[t, pl.ds(16 * w, 16)] = acc[w]
                    sdup_v[t, pl.ds(D + 16 * w, 16)] = acc[w]
                return 0

            lax.fori_loop(0, T, t_body, 0)

            def g_body(t0, acc):
                out = []
                for w in range(4):
                    p = sdup_v[t0, pl.ds(D - 3 + 16 * w, 16)]
                    p = p * sdup_v[t0 + 1, pl.ds(D - 2 + 16 * w, 16)]
                    p = p * sdup_v[t0 + 2, pl.ds(D - 1 + 16 * w, 16)]
                    p = p * sdup_v[t0 + 3, pl.ds(D + 16 * w, 16)]
                    out.append(acc[w] + p)
                return tuple(out)

            hv = lax.fori_loop(0, NWIN, g_body, (zeros,) * 4)
            for w in range(4):
                outb_v[b, pl.ds(16 * w, 16)] = jnp.where(hv[w] > 0.0, 1.0, -1.0)
            return 0

        lax.fori_loop(0, BPW, b_body, 0)
        pltpu.sync_copy(outb_v, out_hbm.at[pl.ds(wid * BPW, BPW)])

    return k(idx, tab)


def kernel(input, level_weight, channels_weight):
    scaled = ((input - LOW) / (HIGH - LOW)) * (NUM_LEVELS - 1)
    lidx = jnp.clip(jnp.round(scaled), 0, NUM_LEVELS - 1).astype(jnp.int32)
    cid = lax.broadcasted_iota(jnp.int32, (B, T, C), 2)
    flatidx = cid * NUM_LEVELS + lidx
    tab3 = _prologue(level_weight, channels_weight)
    tab = tab3.reshape(C * NUM_LEVELS, 2 * D)
    idx = flatidx.reshape(NW, BPW * NSTREAM, SLEN)
    return _sc_encode(idx, tab)


# speedup vs baseline: 22.9463x; 6.3033x over previous
import functools

import jax
import jax.numpy as jnp
from jax import lax
from jax.experimental import pallas as pl
from jax.experimental.pallas import tpu as pltpu
from jax.experimental.pallas import tpu_sc as plsc

NUM_LEVELS = 201
LOW = -100.0
HIGH = 100.0
B, T, C, D = 1024, 20, 26, 64
N = 4
NWIN = T - (N - 1)

NC, NS = 2, 16
NW = NC * NS
NCHUNK = 2
L16 = 16
R = C * NUM_LEVELS
W = D // 4


def _prologue(level_weight_t, channels_weight):
    def body(lwt_ref, cw_ref, tab_ref):
        prod = lwt_ref[...][None, :, :] * cw_ref[...][:, :, None]
        b = (prod + 1.0).astype(jnp.int32)
        packed = (
            b[:, 0:16, :]
            + (b[:, 16:32, :] << 8)
            + (b[:, 32:48, :] << 16)
            + (b[:, 48:64, :] << 24)
        )
        tab_ref[...] = packed

    return pl.pallas_call(
        body,
        out_shape=jax.ShapeDtypeStruct((C, W, NUM_LEVELS), jnp.int32),
    )(level_weight_t, channels_weight)


def _sc_encode(idx, tab):
    mesh = plsc.VectorSubcoreMesh(core_axis_name="c", subcore_axis_name="s")

    @functools.partial(
        pl.kernel,
        mesh=mesh,
        compiler_params=pltpu.CompilerParams(needs_layout_passes=False),
        out_type=jax.ShapeDtypeStruct((B * D,), jnp.float32),
        scratch_types=[
            pltpu.VMEM((R * W,), jnp.int32),
            pltpu.VMEM((T * C * L16,), jnp.int32),
            pltpu.VMEM((T * D * L16,), jnp.float32),
            pltpu.VMEM((L16 * D,), jnp.float32),
        ],
    )
    def k(idx_hbm, tab_hbm, out_hbm, tab_v, idx_v, s_v, outb_v):
        wid = lax.axis_index("s") * NC + lax.axis_index("c")
        pltpu.sync_copy(tab_hbm, tab_v)

        zero_i = jnp.zeros((L16,), jnp.int32)
        zero_f = jnp.zeros((L16,), jnp.float32)
        iota = lax.iota(jnp.int32, L16)

        nchunk_len = T * C * L16
        for ch in range(NCHUNK):
            pltpu.sync_copy(
                idx_hbm.at[
                    pl.ds((wid * NCHUNK + ch) * nchunk_len, nchunk_len)
                ],
                idx_v,
            )

            def t_body(t, _):
                bases = [
                    idx_v[pl.ds((t * C + c) * L16, L16)] for c in range(C)
                ]

                def w_body(w, _):
                    woff = w * NUM_LEVELS
                    acc = zero_i
                    for c in range(C):
                        acc = acc + plsc.load_gather(tab_v, [bases[c] + woff])
                    for kk in range(4):
                        v = ((acc >> (8 * kk)) & 255) - C
                        s_v[pl.ds((t * D + 16 * kk + w) * L16, L16)] = (
                            v.astype(jnp.float32)
                        )
                    return 0

                lax.fori_loop(0, W, w_body, 0)
                return 0

            lax.fori_loop(0, T, t_body, 0)

            def d_body(d, _):
                j3 = (d + (D - 3)) & (D - 1)
                j2 = (d + (D - 2)) & (D - 1)
                j1 = (d + (D - 1)) & (D - 1)

                def g_body(t0, acc):
                    p = s_v[pl.ds((t0 * D + j3) * L16, L16)]
                    p = p * s_v[pl.ds(((t0 + 1) * D + j2) * L16, L16)]
                    p = p * s_v[pl.ds(((t0 + 2) * D + j1) * L16, L16)]
                    p = p * s_v[pl.ds(((t0 + 3) * D + d) * L16, L16)]
                    return acc + p

                hv = lax.fori_loop(0, NWIN, g_body, zero_f)
                o = jnp.where(hv > 0.0, 1.0, -1.0)
                plsc.store_scatter(outb_v, [iota * D + d], o)
                return 0

            lax.fori_loop(0, D, d_body, 0)
            pltpu.sync_copy(
                outb_v,
                out_hbm.at[pl.ds((wid * NCHUNK + ch) * L16 * D, L16 * D)],
            )

    return k(idx, tab)


def kernel(input, level_weight, channels_weight):
    scaled = ((input - LOW) / (HIGH - LOW)) * (NUM_LEVELS - 1)
    lidx = jnp.clip(jnp.round(scaled), 0, NUM_LEVELS - 1).astype(jnp.int32)
    cid = lax.broadcasted_iota(jnp.int32, (B, T, C), 2)
    flatidx = cid * (W * NUM_LEVELS) + lidx
    idx = flatidx.reshape(NW, NCHUNK, L16, T, C).transpose(0, 1, 3, 4, 2)
    idx = idx.reshape(NW * NCHUNK * T * C * L16)
    tab = _prologue(level_weight.T, channels_weight).reshape(R * W)
    return _sc_encode(idx, tab).reshape(B, D)


# speedup vs baseline: 28.2516x; 1.2312x over previous
import functools

import jax
import jax.numpy as jnp
from jax import lax
from jax.experimental import pallas as pl
from jax.experimental.pallas import tpu as pltpu
from jax.experimental.pallas import tpu_sc as plsc

NUM_LEVELS = 201
LOW = -100.0
HIGH = 100.0
B, T, C, D = 1024, 20, 26, 64
N = 4
NWIN = T - (N - 1)

NC, NS = 2, 16
NW = NC * NS
NCHUNK = 2
L16 = 16
R = C * NUM_LEVELS
W = D // 4


def _prologue(level_weight_t, channels_weight):
    def body(lwt_ref, cw_ref, tab_ref):
        prod = lwt_ref[...][None, :, :] * cw_ref[...][:, :, None]
        b = (prod + 1.0).astype(jnp.int32)
        packed = (
            b[:, 0:16, :]
            + (b[:, 16:32, :] << 8)
            + (b[:, 32:48, :] << 16)
            + (b[:, 48:64, :] << 24)
        )
        tab_ref[...] = packed

    return pl.pallas_call(
        body,
        out_shape=jax.ShapeDtypeStruct((C, W, NUM_LEVELS), jnp.int32),
    )(level_weight_t, channels_weight)


def _sc_encode(idx, tab):
    mesh = plsc.VectorSubcoreMesh(core_axis_name="c", subcore_axis_name="s")

    @functools.partial(
        pl.kernel,
        mesh=mesh,
        compiler_params=pltpu.CompilerParams(needs_layout_passes=False),
        out_type=jax.ShapeDtypeStruct((B * D,), jnp.float32),
        scratch_types=[
            pltpu.VMEM((R * W,), jnp.int32),
            pltpu.VMEM((T * C * L16,), jnp.int32),
            pltpu.VMEM((T * D * L16,), jnp.float32),
            pltpu.VMEM((L16 * D,), jnp.float32),
        ],
    )
    def k(idx_hbm, tab_hbm, out_hbm, tab_v, idx_v, s_v, outb_v):
        wid = lax.axis_index("s") * NC + lax.axis_index("c")
        pltpu.sync_copy(tab_hbm, tab_v)

        zero_i = jnp.zeros((L16,), jnp.int32)
        zero_f = jnp.zeros((L16,), jnp.float32)
        iota = lax.iota(jnp.int32, L16)
        iota_tc = iota * (T * C)

        nchunk_len = T * C * L16
        for ch in range(NCHUNK):
            pltpu.sync_copy(
                idx_hbm.at[
                    pl.ds((wid * NCHUNK + ch) * nchunk_len, nchunk_len)
                ],
                idx_v,
            )

            def t_body(t, _):
                bases = [
                    plsc.load_gather(idx_v, [iota_tc + (t * C + c)])
                    for c in range(C)
                ]

                def w_body(w, _):
                    woff = w * NUM_LEVELS
                    acc = zero_i
                    for c in range(C):
                        acc = acc + plsc.load_gather(tab_v, [bases[c] + woff])
                    for kk in range(4):
                        v = ((acc >> (8 * kk)) & 255) - C
                        s_v[pl.ds((t * D + 16 * kk + w) * L16, L16)] = (
                            v.astype(jnp.float32)
                        )
                    return 0

                lax.fori_loop(0, W, w_body, 0)
                return 0

            lax.fori_loop(0, T, t_body, 0)

            def d_body(d, _):
                j3 = (d + (D - 3)) & (D - 1)
                j2 = (d + (D - 2)) & (D - 1)
                j1 = (d + (D - 1)) & (D - 1)

                def g_body(t0, acc):
                    p = s_v[pl.ds((t0 * D + j3) * L16, L16)]
                    p = p * s_v[pl.ds(((t0 + 1) * D + j2) * L16, L16)]
                    p = p * s_v[pl.ds(((t0 + 2) * D + j1) * L16, L16)]
                    p = p * s_v[pl.ds(((t0 + 3) * D + d) * L16, L16)]
                    return acc + p

                hv = lax.fori_loop(0, NWIN, g_body, zero_f)
                o = jnp.where(hv > 0.0, 1.0, -1.0)
                plsc.store_scatter(outb_v, [iota * D + d], o)
                return 0

            lax.fori_loop(0, D, d_body, 0)
            pltpu.sync_copy(
                outb_v,
                out_hbm.at[pl.ds((wid * NCHUNK + ch) * L16 * D, L16 * D)],
            )

    return k(idx, tab)


def kernel(input, level_weight, channels_weight):
    scaled = ((input - LOW) / (HIGH - LOW)) * (NUM_LEVELS - 1)
    lidx = jnp.clip(jnp.round(scaled), 0, NUM_LEVELS - 1).astype(jnp.int32)
    cid = lax.broadcasted_iota(jnp.int32, (B, T, C), 2)
    flatidx = cid * (W * NUM_LEVELS) + lidx
    idx = flatidx.reshape(NW * NCHUNK * T * C * L16)
    tab = _prologue(level_weight.T, channels_weight).reshape(R * W)
    return _sc_encode(idx, tab).reshape(B, D)


# speedup vs baseline: 29.7255x; 1.0522x over previous
import functools

import jax
import jax.numpy as jnp
from jax import lax
from jax.experimental import pallas as pl
from jax.experimental.pallas import tpu as pltpu
from jax.experimental.pallas import tpu_sc as plsc

NUM_LEVELS = 201
LOW = -100.0
HIGH = 100.0
B, T, C, D = 1024, 20, 26, 64
N = 4
NWIN = T - (N - 1)

NC, NS = 2, 16
NW = NC * NS
NCHUNK = 2
L16 = 16
R = C * NUM_LEVELS
W = D // 4


def _prologue(level_weight_t, channels_weight):
    def body(lwt_ref, cw_ref, tab_ref):
        prod = lwt_ref[...][None, :, :] * cw_ref[...][:, :, None]
        b = (prod + 1.0).astype(jnp.int32)
        packed = (
            b[:, 0:16, :]
            + (b[:, 16:32, :] << 8)
            + (b[:, 32:48, :] << 16)
            + (b[:, 48:64, :] << 24)
        )
        tab_ref[...] = packed

    return pl.pallas_call(
        body,
        out_shape=jax.ShapeDtypeStruct((C, W, NUM_LEVELS), jnp.int32),
    )(level_weight_t, channels_weight)


def _sc_encode(idx, tab):
    mesh = plsc.VectorSubcoreMesh(core_axis_name="c", subcore_axis_name="s")

    @functools.partial(
        pl.kernel,
        mesh=mesh,
        compiler_params=pltpu.CompilerParams(needs_layout_passes=False),
        out_type=jax.ShapeDtypeStruct((B * D,), jnp.float32),
        scratch_types=[
            pltpu.VMEM((R * W,), jnp.int32),
            pltpu.VMEM((T * C * L16,), jnp.int32),
            pltpu.VMEM((T * D * L16,), jnp.float32),
            pltpu.VMEM((L16 * D,), jnp.float32),
        ],
    )
    def k(idx_hbm, tab_hbm, out_hbm, tab_v, idx_v, s_v, outb_v):
        wid = lax.axis_index("s") * NC + lax.axis_index("c")
        pltpu.sync_copy(tab_hbm, tab_v)

        zero_i = jnp.zeros((L16,), jnp.int32)
        zero_f = jnp.zeros((L16,), jnp.float32)
        iota = lax.iota(jnp.int32, L16)
        iota_tc = iota * (T * C)

        nchunk_len = T * C * L16
        for ch in range(NCHUNK):
            pltpu.sync_copy(
                idx_hbm.at[
                    pl.ds((wid * NCHUNK + ch) * nchunk_len, nchunk_len)
                ],
                idx_v,
            )

            def t_body(t, _):
                bases = [
                    plsc.load_gather(idx_v, [iota_tc + (t * C + c)])
                    for c in range(C)
                ]

                def w_body(w, _):
                    woff = w * NUM_LEVELS
                    vals = [
                        plsc.load_gather(tab_v, [bases[c] + woff])
                        for c in range(C)
                    ]
                    while len(vals) > 1:
                        vals = [
                            vals[i] + vals[i + 1]
                            for i in range(0, len(vals) - 1, 2)
                        ] + ([vals[-1]] if len(vals) % 2 else [])
                    acc = vals[0]
                    for kk in range(4):
                        v = ((acc >> (8 * kk)) & 255) - C
                        s_v[pl.ds((t * D + 16 * kk + w) * L16, L16)] = (
                            v.astype(jnp.float32)
                        )
                    return 0

                lax.fori_loop(0, W, w_body, 0)
                return 0

            lax.fori_loop(0, T, t_body, 0)

            def d_body(d, _):
                j3 = (d + (D - 3)) & (D - 1)
                j2 = (d + (D - 2)) & (D - 1)
                j1 = (d + (D - 1)) & (D - 1)

                prods = []
                for t0 in range(NWIN):
                    p = s_v[pl.ds((t0 * D + j3) * L16, L16)]
                    p = p * s_v[pl.ds(((t0 + 1) * D + j2) * L16, L16)]
                    p = p * s_v[pl.ds(((t0 + 2) * D + j1) * L16, L16)]
                    p = p * s_v[pl.ds(((t0 + 3) * D + d) * L16, L16)]
                    prods.append(p)
                while len(prods) > 1:
                    prods = [
                        prods[i] + prods[i + 1]
                        for i in range(0, len(prods) - 1, 2)
                    ] + ([prods[-1]] if len(prods) % 2 else [])
                hv = prods[0]
                o = jnp.where(hv > 0.0, 1.0, -1.0)
                plsc.store_scatter(outb_v, [iota * D + d], o)
                return 0

            lax.fori_loop(0, D, d_body, 0)
            pltpu.sync_copy(
                outb_v,
                out_hbm.at[pl.ds((wid * NCHUNK + ch) * L16 * D, L16 * D)],
            )

    return k(idx, tab)


def kernel(input, level_weight, channels_weight):
    scaled = ((input - LOW) / (HIGH - LOW)) * (NUM_LEVELS - 1)
    lidx = jnp.clip(jnp.round(scaled), 0, NUM_LEVELS - 1).astype(jnp.int32)
    cid = lax.broadcasted_iota(jnp.int32, (B, T, C), 2)
    flatidx = cid * (W * NUM_LEVELS) + lidx
    idx = flatidx.reshape(NW * NCHUNK * T * C * L16)
    tab = _prologue(level_weight.T, channels_weight).reshape(R * W)
    return _sc_encode(idx, tab).reshape(B, D)
